# baseline (speedup 1.0000x reference)
import functools

import jax
import jax.numpy as jnp
from jax import lax
from jax.experimental import pallas as pl
from jax.experimental.pallas import tpu as pltpu
from jax.experimental.pallas import tpu_sc as plsc

_CAPACITY = 0.5
_WINDOW = 100

_RTC = 10240
_NW = 32


def _dist_kernel(o_ref, p_ref, r_ref, dst_ref, dch_ref):
    o = o_ref[...]
    p = p_ref[...]
    r = r_ref[...]
    dst_ref[0, 0, :] = jnp.mean((p - o) ** 2, axis=-1)
    dch_ref[0, 0, :] = jnp.mean((p - r) ** 2, axis=-1)


_G = 16


def _sc_dist_kernel(o_hbm, p_hbm, r_hbm, dst_hbm, dch_hbm,
                    buf, accst, accch, stbuf, chbuf, sem0, sem1, *,
                    rtc, rpw, d):
    wid = lax.axis_index("s") * 2 + lax.axis_index("c")
    base = rtc + wid * rpw
    lane = lax.iota(jnp.int32, 16)
    lane16 = lane * _G
    inv_d = jnp.float32(1.0 / d)
    zero = jnp.zeros((16,), jnp.float32)
    ngroups = rpw // _G

    SR = 8
    stage_words = 3 * SR * d

    def start(s, b):
        row0 = base + s * SR
        sem = sem0 if b == 0 else sem1
        for r in range(SR):
            for ai, hbm in enumerate((o_hbm, p_hbm, r_hbm)):
                pltpu.async_copy(
                    hbm.at[row0 + r],
                    buf.at[pl.ds(b * stage_words + (r * 3 + ai) * d, d)],
                    sem)

    def wait(b):
        sem = sem0 if b == 0 else sem1
        for r in range(SR):
            for ai, hbm in enumerate((o_hbm, p_hbm, r_hbm)):
                pltpu.make_async_copy(
                    hbm.at[base],
                    buf.at[pl.ds(b * stage_words + (r * 3 + ai) * d, d)],
                    sem).wait()

    nstages = rpw // SR
    start(0, 0)
    start(1, 1)

    def group(g, carry):
        for h in range(2):
            b = h
            s = g * 2 + h
            wait(b)
            for r8 in range(SR):
                ob = b * stage_words + (r8 * 3 + 0) * d
                pb = b * stage_words + (r8 * 3 + 1) * d
                rb = b * stage_words + (r8 * 3 + 2) * d

                @plsc.parallel_loop(0, d // 32, 1, unroll=4,
                                    carry=(zero, zero, zero, zero))
                def acc_loop(i, acc, _ob=ob, _pb=pb, _rb=rb):
                    a0, a1, c0, c1 = acc
                    off = i * 32
                    for k in (0, 16):
                        o_v = buf[pl.ds(_ob + off + k, 16)]
                        p_v = buf[pl.ds(_pb + off + k, 16)]
                        r_v = buf[pl.ds(_rb + off + k, 16)]
                        d0 = p_v - o_v
                        d1 = p_v - r_v
                        if k == 0:
                            a0 = a0 + d0 * d0
                            c0 = c0 + d1 * d1
                        else:
                            a1 = a1 + d0 * d0
                            c1 = c1 + d1 * d1
                    return a0, a1, c0, c1

                a0, a1, c0, c1 = acc_loop
                r = h * SR + r8
                accst[pl.ds(r * 16, 16)] = a0 + a1
                accch[pl.ds(r * 16, 16)] = c0 + c1

            nxt = s + 2

            @pl.when(nxt < nstages)
            def _():
                start(nxt, b)

        res_st = zero
        res_ch = zero
        for c in range(16):
            res_st = res_st + plsc.load_gather(accst, [lane16 + c])
            res_ch = res_ch + plsc.load_gather(accch, [lane16 + c])
        stbuf[...] = res_st * inv_d
        chbuf[...] = res_ch * inv_d
        off = wid * rpw + g * _G
        pltpu.sync_copy(stbuf, dst_hbm.at[pl.ds(off, _G)])
        pltpu.sync_copy(chbuf, dch_hbm.at[pl.ds(off, _G)])
        return carry

    lax.fori_loop(0, ngroups, group, 0)


def _route_kernel(dst_ref, dch_ref, par_ref, mask_ref, sig_ref, sce_ref,
                  scu_ref, *, target):
    d_st = dst_ref[...]
    d_ch = dch_ref[...]
    sp_ce = par_ref[0]
    sp_cu = par_ref[1]
    cu_mult = par_ref[2]
    log_off = par_ref[3]
    B, S = d_st.shape
    w = _WINDOW
    padded = jnp.concatenate(
        [jnp.broadcast_to(d_st[:, :1], (B, w - 1)), d_st], axis=1)
    c = padded
    sh = 1
    while sh < padded.shape[1]:
        z = jnp.zeros((B, sh), c.dtype)
        c = c + jnp.concatenate([z, c[:, :-sh]], axis=1)
        sh *= 2
    c = jnp.concatenate([jnp.zeros((B, 1), c.dtype), c], axis=1)
    ma = (c[:, w:] - c[:, :-w]) / jnp.float32(w)

    ce_val = d_st - (d_ch - log_off)
    cu_val = d_st - cu_mult * ma
    s_ce = 1.0 / (1.0 + jnp.exp(-(sp_ce * ce_val)))
    s_cu = 1.0 / (1.0 + jnp.exp(-(sp_cu * cu_val)))
    signal = s_ce + s_cu - s_ce * s_cu

    if target is None:
        mask_ref[...] = jnp.ones_like(signal)
    else:
        u = lax.bitcast_convert_type(signal, jnp.int32)
        imin = jnp.int32(-(2 ** 31))
        ks = jnp.where(u >= 0, u, (~u) ^ imin)
        ans = jnp.int32(0)
        for b in range(31, -1, -1):
            bit = imin if b == 31 else jnp.int32(1 << b)
            cand = ans | bit
            cnt = jnp.sum((ks < (cand ^ imin)).astype(jnp.int32))
            ans = jnp.where(cnt <= target, cand, ans)
        tbits = jnp.where(ans < 0, ans ^ imin, ~ans)
        thr = lax.bitcast_convert_type(tbits, jnp.float32)
        mask_ref[...] = (signal >= thr).astype(jnp.float32)
    sig_ref[...] = signal
    sce_ref[...] = s_ce
    scu_ref[...] = s_cu


def kernel(original, posterior, prior, beta_ce, beta_cu, cu_mult, ce_offset):
    B, S, D = original.shape
    BS = B * S
    o2 = original.reshape(BS, D)
    p2 = posterior.reshape(BS, D)
    r2 = prior.reshape(BS, D)

    rtc = _RTC
    SB = 512
    ntc = rtc // SB
    rows_sc = BS - rtc
    rpw = rows_sc // _NW
    mesh = plsc.VectorSubcoreMesh(core_axis_name="c", subcore_axis_name="s")
    sc_call = functools.partial(
        pl.kernel,
        mesh=mesh,
        out_type=[jax.ShapeDtypeStruct((rows_sc,), jnp.float32)] * 2,
        scratch_types=[
            pltpu.VMEM((2 * 3 * 8 * D,), jnp.float32),
            pltpu.VMEM((16 * _G,), jnp.float32),
            pltpu.VMEM((16 * _G,), jnp.float32),
            pltpu.VMEM((16,), jnp.float32),
            pltpu.VMEM((16,), jnp.float32),
            pltpu.SemaphoreType.DMA,
            pltpu.SemaphoreType.DMA,
        ],
        compiler_params=pltpu.CompilerParams(needs_layout_passes=False),
    )(functools.partial(_sc_dist_kernel, rtc=rtc, rpw=rpw, d=D))
    dst_sc, dch_sc = sc_call(o2, p2, r2)

    dst_tc, dch_tc = pl.pallas_call(
        _dist_kernel,
        grid=(ntc,),
        in_specs=[pl.BlockSpec((SB, D), lambda s: (s, 0))] * 3,
        out_specs=[pl.BlockSpec((1, 1, SB), lambda s: (s, 0, 0))] * 2,
        out_shape=[jax.ShapeDtypeStruct((ntc, 1, SB), jnp.float32)] * 2,
    )(o2, p2, r2)

    d_st = jnp.concatenate([dst_tc.reshape(rtc), dst_sc]).reshape(B, S)
    d_ch = jnp.concatenate([dch_tc.reshape(rtc), dch_sc]).reshape(B, S)

    params = jnp.stack([
        jax.nn.softplus(jnp.asarray(beta_ce, jnp.float32)),
        jax.nn.softplus(jnp.asarray(beta_cu, jnp.float32)),
        jnp.asarray(cu_mult, jnp.float32),
        jnp.log(jnp.asarray(ce_offset, jnp.float32) + 1e-10),
    ])

    n = BS
    k = int(_CAPACITY * n)
    target = (n - k) if k < n else None
    mask, signal, s_ce, s_cu = pl.pallas_call(
        functools.partial(_route_kernel, target=target),
        in_specs=[
            pl.BlockSpec((B, S), lambda: (0, 0)),
            pl.BlockSpec((B, S), lambda: (0, 0)),
            pl.BlockSpec(memory_space=pltpu.SMEM),
        ],
        out_specs=[pl.BlockSpec((B, S), lambda: (0, 0))] * 4,
        out_shape=[jax.ShapeDtypeStruct((B, S), jnp.float32)] * 4,
    )(d_st, d_ch, params)
    return mask, signal, s_ce, s_cu


# speedup vs baseline: 1.1743x; 1.1743x over previous
import functools

import jax
import jax.numpy as jnp
from jax import lax
from jax.experimental import pallas as pl
from jax.experimental.pallas import tpu as pltpu

_CAPACITY = 0.5
_WINDOW = 100


def _dist_kernel(o_ref, p_ref, r_ref, dst_ref, dch_ref):
    o = o_ref[0]
    p = p_ref[0]
    r = r_ref[0]
    dst_ref[0, 0, 0, :] = jnp.mean((p - o) ** 2, axis=-1)
    dch_ref[0, 0, 0, :] = jnp.mean((p - r) ** 2, axis=-1)


def _route_kernel(dst_ref, dch_ref, par_ref, mask_ref, sig_ref, sce_ref,
                  scu_ref, *, target):
    d_st = dst_ref[...]
    d_ch = dch_ref[...]
    sp_ce = par_ref[0]
    sp_cu = par_ref[1]
    cu_mult = par_ref[2]
    log_off = par_ref[3]
    B, S = d_st.shape
    w = _WINDOW
    padded = jnp.concatenate(
        [jnp.broadcast_to(d_st[:, :1], (B, w - 1)), d_st], axis=1)
    c = padded
    sh = 1
    while sh < padded.shape[1]:
        z = jnp.zeros((B, sh), c.dtype)
        c = c + jnp.concatenate([z, c[:, :-sh]], axis=1)
        sh *= 2
    c = jnp.concatenate([jnp.zeros((B, 1), c.dtype), c], axis=1)
    ma = (c[:, w:] - c[:, :-w]) / jnp.float32(w)

    ce_val = d_st - (d_ch - log_off)
    cu_val = d_st - cu_mult * ma
    s_ce = 1.0 / (1.0 + jnp.exp(-(sp_ce * ce_val)))
    s_cu = 1.0 / (1.0 + jnp.exp(-(sp_cu * cu_val)))
    signal = s_ce + s_cu - s_ce * s_cu

    if target is None:
        mask_ref[...] = jnp.ones_like(signal)
    else:
        u = lax.bitcast_convert_type(signal, jnp.int32)
        imin = jnp.int32(-(2 ** 31))
        ks = jnp.where(u >= 0, u, (~u) ^ imin)
        ans = jnp.int32(0)
        for b in range(30, -2, -2):
            cnt_le = []
            for q in (1, 2, 3):
                qb = (q << b) & 0xFFFFFFFF
                qb = qb - (1 << 32) if qb >= (1 << 31) else qb
                cand = ans | jnp.int32(qb)
                cnt = jnp.sum((ks < (cand ^ imin)).astype(jnp.int32))
                cnt_le.append((cnt <= target).astype(jnp.int32))
            qstar = cnt_le[0] + cnt_le[1] + cnt_le[2]
            ans = ans | lax.shift_left(qstar, jnp.int32(b))
        tbits = jnp.where(ans < 0, ans ^ imin, ~ans)
        thr = lax.bitcast_convert_type(tbits, jnp.float32)
        mask_ref[...] = (signal >= thr).astype(jnp.float32)
    sig_ref[...] = signal
    sce_ref[...] = s_ce
    scu_ref[...] = s_cu


def kernel(original, posterior, prior, beta_ce, beta_cu, cu_mult, ce_offset):
    B, S, D = original.shape
    SB = 512
    NS = S // SB
    dst4, dch4 = pl.pallas_call(
        _dist_kernel,
        grid=(B, NS),
        in_specs=[pl.BlockSpec((1, SB, D), lambda b, s: (b, s, 0))] * 3,
        out_specs=[pl.BlockSpec((1, 1, 1, SB), lambda b, s: (b, s, 0, 0))] * 2,
        out_shape=[jax.ShapeDtypeStruct((B, NS, 1, SB), jnp.float32)] * 2,
    )(original, posterior, prior)
    d_st = dst4.reshape(B, S)
    d_ch = dch4.reshape(B, S)

    params = jnp.stack([
        jax.nn.softplus(jnp.asarray(beta_ce, jnp.float32)),
        jax.nn.softplus(jnp.asarray(beta_cu, jnp.float32)),
        jnp.asarray(cu_mult, jnp.float32),
        jnp.log(jnp.asarray(ce_offset, jnp.float32) + 1e-10),
    ])

    n = B * S
    k = int(_CAPACITY * n)
    target = (n - k) if k < n else None
    mask, signal, s_ce, s_cu = pl.pallas_call(
        functools.partial(_route_kernel, target=target),
        in_specs=[
            pl.BlockSpec((B, S), lambda: (0, 0)),
            pl.BlockSpec((B, S), lambda: (0, 0)),
            pl.BlockSpec(memory_space=pltpu.SMEM),
        ],
        out_specs=[pl.BlockSpec((B, S), lambda: (0, 0))] * 4,
        out_shape=[jax.ShapeDtypeStruct((B, S), jnp.float32)] * 4,
    )(d_st, d_ch, params)
    return mask, signal, s_ce, s_cu
